# Initial kernel scaffold; baseline (speedup 1.0000x reference)
import jax
import jax.numpy as jnp
from jax.experimental import pallas as pl


def kernel(magno_patches, vit_positional_embedding, scores):
    raise NotImplementedError("write your pallas kernel here")


# speedup vs baseline: 1.8041x; 1.8041x over previous
import jax
import jax.numpy as jnp
from jax import lax
from jax.experimental import pallas as pl


def _body(scores_ref, patches_ref, pos_ref, out_ref):
    N = scores_ref.shape[-1]
    K = out_ref.shape[1]
    s = scores_ref[0]
    scol = jnp.reshape(s, (N, 1))
    ni = lax.broadcasted_iota(jnp.int32, (N, N), 0)
    mi = lax.broadcasted_iota(jnp.int32, (N, N), 1)
    beats = (s > scol) | ((s == scol) & (mi < ni))
    rank = jnp.sum(beats.astype(jnp.int32), axis=1, keepdims=True)
    jrow = lax.broadcasted_iota(jnp.int32, (1, K), 1)
    onehot = (rank == jrow).astype(jnp.float32)
    summed = patches_ref[0] + pos_ref[...]
    out_ref[0] = lax.dot_general(
        onehot, summed,
        dimension_numbers=(((0,), (0,)), ((), ())),
        preferred_element_type=jnp.float32,
    )


def kernel(magno_patches, vit_positional_embedding, scores):
    B, N, D = magno_patches.shape
    K = N // 4
    pos = vit_positional_embedding[0, 1:, :]
    scores3 = scores.reshape(B, 1, N)
    return pl.pallas_call(
        _body,
        grid=(B,),
        in_specs=[
            pl.BlockSpec((1, 1, N), lambda b: (b, 0, 0)),
            pl.BlockSpec((1, N, D), lambda b: (b, 0, 0)),
            pl.BlockSpec((N, D), lambda b: (0, 0)),
        ],
        out_specs=pl.BlockSpec((1, K, D), lambda b: (b, 0, 0)),
        out_shape=jax.ShapeDtypeStruct((B, K, D), jnp.float32),
    )(scores3, magno_patches, pos)
